# baseline (speedup 1.0000x reference)
import functools

import jax
import jax.numpy as jnp
from jax import lax
from jax.experimental import pallas as pl
from jax.experimental.pallas import tpu as pltpu
from jax.experimental.pallas import tpu_sc as plsc

N_NODES = 10000
N_EDGES = 320000
Z_DIM = 128

NUM_CORES = 2
NUM_SUBCORES = 16
NUM_WORKERS = NUM_CORES * NUM_SUBCORES
EPW = N_EDGES // NUM_WORKERS
CHUNK = 80
CPW = EPW // CHUNK
STRIPE = 632
STRIPE_LAST = N_NODES - (NUM_SUBCORES - 1) * STRIPE

SELU_LAM = 1.0507009873554805
SELU_ALPHA = 1.6732632423543772

_SC_PARAMS = pltpu.CompilerParams(
    needs_layout_passes=False, use_tc_tiling_on_sc=False,
    internal_scratch_in_bytes=0)


def _stage_scores(z, wr):
    def body(z_ref, w_ref, s_ref):
        s_ref[...] = lax.dot_general(
            w_ref[...], z_ref[...], (((1,), (1,)), ((), ())),
            preferred_element_type=jnp.float32)

    return pl.pallas_call(
        body,
        out_shape=jax.ShapeDtypeStruct((NUM_CORES, N_NODES), jnp.float32),
    )(z, wr)


def _sc_edge_weights(s_pair, src_w, dst_w):
    mesh = plsc.VectorSubcoreMesh(core_axis_name="c", subcore_axis_name="s")

    @functools.partial(
        pl.kernel,
        out_type=[
            jax.ShapeDtypeStruct((NUM_WORKERS, EPW), jnp.float32),
            jax.ShapeDtypeStruct((NUM_CORES, NUM_SUBCORES, N_NODES),
                                 jnp.float32),
        ],
        mesh=mesh,
        compiler_params=_SC_PARAMS,
        scratch_types=[
            pltpu.VMEM((N_NODES,), jnp.float32),
            pltpu.VMEM((N_NODES,), jnp.float32),
            pltpu.VMEM((1, EPW), jnp.int32),
            pltpu.VMEM((1, EPW), jnp.int32),
            pltpu.VMEM((1, EPW), jnp.float32),
            pltpu.VMEM((N_NODES,), jnp.float32),
        ],
    )
    def run(s_hbm, src_hbm, dst_hbm, ex_hbm, den_hbm,
            s1_v, s2_v, srcv, dstv, exv, denv):
        cid = lax.axis_index("c")
        sid = lax.axis_index("s")
        wid = sid * NUM_CORES + cid

        pltpu.sync_copy(s_hbm.at[0], s1_v)
        pltpu.sync_copy(s_hbm.at[1], s2_v)
        pltpu.sync_copy(src_hbm.at[wid], srcv.at[0])
        pltpu.sync_copy(dst_hbm.at[wid], dstv.at[0])

        zeros16 = jnp.zeros((16,), jnp.float32)

        @pl.loop(0, N_NODES // 16)
        def _zero(i):
            denv[pl.ds(i * 16, 16)] = zeros16

        @pl.loop(0, EPW // 16)
        def _edge(g):
            sl = pl.ds(g * 16, 16)
            sidx = srcv[0, sl]
            didx = dstv[0, sl]
            x = plsc.load_gather(s1_v, [sidx]) + plsc.load_gather(s2_v, [didx])
            selu = jnp.where(
                x > 0, SELU_LAM * x,
                (SELU_LAM * SELU_ALPHA) * (jnp.exp(x) - 1.0))
            ex = jnp.exp(selu)
            exv[0, sl] = ex
            plsc.addupdate_scatter(denv, [didx], ex)

        pltpu.sync_copy(exv.at[0], ex_hbm.at[wid])
        pltpu.sync_copy(denv, den_hbm.at[cid, sid])

    return run(s_pair, src_w, dst_w)


def _sc_edge_pass(z_bf, z_f32, src_w, dst_w, ex_w):
    mesh = plsc.VectorSubcoreMesh(core_axis_name="c", subcore_axis_name="s")
    HALF = 62

    @functools.partial(
        pl.kernel,
        out_type=jax.ShapeDtypeStruct((NUM_CORES, N_NODES, Z_DIM),
                                      jnp.float32),
        mesh=mesh,
        compiler_params=_SC_PARAMS,
        scratch_types=[
            pltpu.VMEM((64, CHUNK), jnp.int32),
            pltpu.VMEM((64, CHUNK), jnp.int32),
            pltpu.VMEM((64, CHUNK), jnp.float32),
            pltpu.VMEM((CHUNK, Z_DIM), jnp.bfloat16),
            pltpu.VMEM((CHUNK, Z_DIM), jnp.bfloat16),
            pltpu.VMEM((CHUNK, Z_DIM), jnp.float32),
            pltpu.VMEM((CHUNK, Z_DIM), jnp.float32),
            pltpu.VMEM_SHARED((N_NODES, Z_DIM), jnp.float32),
            pltpu.SemaphoreType.DMA,
            pltpu.SemaphoreType.DMA,
            pltpu.SemaphoreType.DMA,
            pltpu.SemaphoreType.DMA,
        ],
    )
    def run(zb_hbm, zf_hbm, src_hbm, dst_hbm, ex_hbm, out_hbm,
            srcv, dstv, exv, gb0, gb1, fb0, fb1, table,
            gsem0, gsem1, ssem0, ssem1):
        cid = lax.axis_index("c")
        sid = lax.axis_index("s")
        wid = sid * NUM_CORES + cid

        gb = (gb0, gb1)
        fb = (fb0, fb1)
        gsem = (gsem0, gsem1)
        ssem = (ssem0, ssem1)

        pltpu.sync_copy(src_hbm.at[wid, pl.ds(0, 64)], srcv)
        pltpu.sync_copy(dst_hbm.at[wid, pl.ds(0, 64)], dstv)
        pltpu.sync_copy(ex_hbm.at[wid, pl.ds(0, 64)], exv)

        zeros16 = jnp.zeros((16,), jnp.float32)

        @pl.loop(0, CHUNK)
        def _zrows(i):
            for j in range(Z_DIM // 16):
                fb0[i, pl.ds(j * 16, 16)] = zeros16

        @pl.when(sid < NUM_SUBCORES - 1)
        def _zstripe():
            for k in range(STRIPE // CHUNK):
                pltpu.sync_copy(
                    fb0, table.at[pl.ds(sid * STRIPE + k * CHUNK, CHUNK)])
            rem = STRIPE % CHUNK
            pltpu.sync_copy(
                fb0.at[pl.ds(0, rem)],
                table.at[pl.ds(sid * STRIPE + (STRIPE // CHUNK) * CHUNK, rem)])

        @pl.when(sid == NUM_SUBCORES - 1)
        def _zstripe_last():
            base = (NUM_SUBCORES - 1) * STRIPE
            for k in range(STRIPE_LAST // CHUNK):
                pltpu.sync_copy(
                    fb0, table.at[pl.ds(base + k * CHUNK, CHUNK)])
            rem = STRIPE_LAST % CHUNK
            pltpu.sync_copy(
                fb0.at[pl.ds(0, rem)],
                table.at[pl.ds(base + (STRIPE_LAST // CHUNK) * CHUNK, rem)])

        plsc.subcore_barrier()

        def gather_start(row, b):
            pltpu.async_copy(zb_hbm.at[srcv.at[row]], gb[b], gsem[b])

        def gather_wait(b):
            pltpu.make_async_copy(
                zb_hbm.at[srcv.at[0]], gb[b], gsem[b]).wait()

        def scatter_start(row, b):
            pltpu.async_copy(
                fb[b], table.at[dstv.at[row]], ssem[b], add=True)

        def scatter_wait(b):
            pltpu.make_async_copy(
                zf_hbm.at[pl.ds(0, CHUNK)], fb[b], ssem[b]).wait()

        def compute(row, b):
            gbuf = gb[b]
            fbuf = fb[b]
            for g in range(CHUNK // 16):
                ex16 = exv[row, pl.ds(g * 16, 16)]
                for i in range(16):
                    a = ex16[i]
                    r = g * 16 + i
                    for j in range(Z_DIM // 32):
                        v = gbuf[r, pl.ds(j * 32, 32)]
                        lo, hi = plsc.unpack(v, format=plsc.PackFormat.INTERLEAVED)
                        fbuf[r, pl.ds(j * 32, 16)] = lo * a
                        fbuf[r, pl.ds(j * 32 + 16, 16)] = hi * a

        gather_start(0, 0)

        @pl.loop(0, HALF // 2)
        def _pair(i):
            r0 = i * 2
            gather_wait(0)

            @pl.when(i > 0)
            def _():
                scatter_wait(1)

            gather_start(r0 + 1, 1)
            compute(r0, 0)
            scatter_start(r0, 0)

            gather_wait(1)
            scatter_wait(0)
            gather_start(r0 + 2, 0)
            compute(r0 + 1, 1)
            scatter_start(r0 + 1, 1)

        gather_wait(0)
        scatter_wait(1)
        pltpu.sync_copy(src_hbm.at[wid, pl.ds(HALF, CPW - HALF)],
                        srcv.at[pl.ds(0, CPW - HALF)])
        pltpu.sync_copy(dst_hbm.at[wid, pl.ds(HALF, CPW - HALF)],
                        dstv.at[pl.ds(0, CPW - HALF)])
        pltpu.sync_copy(ex_hbm.at[wid, pl.ds(HALF, CPW - HALF)],
                        exv.at[pl.ds(0, CPW - HALF)])

        gather_start(1, 1)
        compute(0, 0)
        scatter_start(0, 0)

        @pl.loop(0, (CPW - HALF - 1) // 2)
        def _pair2(i):
            ra = 1 + i * 2
            gather_wait(1)
            scatter_wait(0)
            gather_start(ra + 1, 0)
            compute(ra, 1)
            scatter_start(ra, 1)

            gather_wait(0)
            scatter_wait(1)

            @pl.when(i < (CPW - HALF - 1) // 2 - 1)
            def _():
                gather_start(ra + 2, 1)

            compute(ra + 1, 0)
            scatter_start(ra + 1, 0)

        scatter_wait(0)

        plsc.subcore_barrier()

        @pl.when(sid < NUM_SUBCORES - 1)
        def _dump():
            pltpu.sync_copy(
                table.at[pl.ds(sid * STRIPE, STRIPE)],
                out_hbm.at[cid, pl.ds(sid * STRIPE, STRIPE)])

        @pl.when(sid == NUM_SUBCORES - 1)
        def _dump_last():
            base = (NUM_SUBCORES - 1) * STRIPE
            pltpu.sync_copy(
                table.at[pl.ds(base, STRIPE_LAST)],
                out_hbm.at[cid, pl.ds(base, STRIPE_LAST)])

    return run(z_bf, z_f32, src_w, dst_w, ex_w)


def _stage_combine(agg2, dens, z):
    def body(agg_ref, den_ref, z_ref, out_ref):
        acc = agg_ref[0] + agg_ref[1]
        denom = jnp.sum(den_ref[...], axis=(0, 1))[:, None]
        out_ref[...] = jnp.where(denom > 0, acc / denom, z_ref[...])

    return pl.pallas_call(
        body,
        out_shape=jax.ShapeDtypeStruct((N_NODES, Z_DIM), jnp.float32),
    )(agg2, dens, z)


_PERM = tuple(32 * g + (k // 2 if k % 2 == 0 else 16 + k // 2)
              for g in range(Z_DIM // 32) for k in range(32))


@jax.jit
def kernel(z, edge_index, W):
    wr = W.reshape(NUM_CORES, Z_DIM)
    src_w = edge_index[0].reshape(NUM_WORKERS, EPW)
    dst_w = edge_index[1].reshape(NUM_WORKERS, EPW)
    z_bf = z[:, jnp.asarray(_PERM, dtype=jnp.int32)].astype(jnp.bfloat16)
    s_pair = _stage_scores(z, wr)
    ex_w, dens = _sc_edge_weights(s_pair, src_w, dst_w)
    agg2 = _sc_edge_pass(
        z_bf, z,
        src_w.reshape(NUM_WORKERS, CPW, CHUNK),
        dst_w.reshape(NUM_WORKERS, CPW, CHUNK),
        ex_w.reshape(NUM_WORKERS, CPW, CHUNK),
        )
    return _stage_combine(agg2, dens, z)


# speedup vs baseline: 1.0239x; 1.0239x over previous
import functools

import jax
import jax.numpy as jnp
from jax import lax
from jax.experimental import pallas as pl
from jax.experimental.pallas import tpu as pltpu
from jax.experimental.pallas import tpu_sc as plsc

N_NODES = 10000
N_EDGES = 320000
Z_DIM = 128

NUM_CORES = 2
NUM_SUBCORES = 16
NUM_WORKERS = NUM_CORES * NUM_SUBCORES
EPW = N_EDGES // NUM_WORKERS
CHUNK = 80
CPW = EPW // CHUNK
STRIPE = 632
STRIPE_LAST = N_NODES - (NUM_SUBCORES - 1) * STRIPE

SELU_LAM = 1.0507009873554805
SELU_ALPHA = 1.6732632423543772

_SC_PARAMS = pltpu.CompilerParams(
    needs_layout_passes=False, use_tc_tiling_on_sc=False,
    internal_scratch_in_bytes=0)


def _stage_scores(z, wr, perm):
    def body(z_ref, w_ref, p_ref, s_ref, zb_ref):
        zb = z_ref[...]
        s_ref[...] = lax.dot_general(
            w_ref[...], zb, (((1,), (1,)), ((), ())),
            preferred_element_type=jnp.float32)
        idx = jnp.broadcast_to(p_ref[...][None, :], zb.shape)
        zb_ref[...] = jnp.take_along_axis(zb, idx, axis=1).astype(jnp.bfloat16)

    return pl.pallas_call(
        body,
        out_shape=[
            jax.ShapeDtypeStruct((NUM_CORES, N_NODES), jnp.float32),
            jax.ShapeDtypeStruct((N_NODES, Z_DIM), jnp.bfloat16),
        ],
    )(z, wr, perm)


def _sc_edge_weights(s_pair, src_w, dst_w):
    mesh = plsc.VectorSubcoreMesh(core_axis_name="c", subcore_axis_name="s")

    @functools.partial(
        pl.kernel,
        out_type=[
            jax.ShapeDtypeStruct((NUM_WORKERS, EPW), jnp.float32),
            jax.ShapeDtypeStruct((NUM_CORES, NUM_SUBCORES, N_NODES),
                                 jnp.float32),
        ],
        mesh=mesh,
        compiler_params=_SC_PARAMS,
        scratch_types=[
            pltpu.VMEM((N_NODES,), jnp.float32),
            pltpu.VMEM((N_NODES,), jnp.float32),
            pltpu.VMEM((1, EPW), jnp.int32),
            pltpu.VMEM((1, EPW), jnp.int32),
            pltpu.VMEM((1, EPW), jnp.float32),
            pltpu.VMEM((N_NODES,), jnp.float32),
            pltpu.SemaphoreType.DMA,
        ],
    )
    def run(s_hbm, src_hbm, dst_hbm, ex_hbm, den_hbm,
            s1_v, s2_v, srcv, dstv, exv, denv, sem):
        cid = lax.axis_index("c")
        sid = lax.axis_index("s")
        wid = sid * NUM_CORES + cid

        pltpu.async_copy(s_hbm.at[0], s1_v, sem)
        pltpu.async_copy(s_hbm.at[1], s2_v, sem)
        pltpu.async_copy(src_hbm.at[wid], srcv.at[0], sem)
        pltpu.async_copy(dst_hbm.at[wid], dstv.at[0], sem)

        zeros16 = jnp.zeros((16,), jnp.float32)

        @pl.loop(0, N_NODES // 16)
        def _zero(i):
            denv[pl.ds(i * 16, 16)] = zeros16

        pltpu.make_async_copy(s_hbm.at[0], s1_v, sem).wait()
        pltpu.make_async_copy(s_hbm.at[1], s2_v, sem).wait()
        pltpu.make_async_copy(src_hbm.at[wid], srcv.at[0], sem).wait()
        pltpu.make_async_copy(dst_hbm.at[wid], dstv.at[0], sem).wait()

        def edge_group(g):
            sl = pl.ds(g * 16, 16)
            sidx = srcv[0, sl]
            didx = dstv[0, sl]
            x = plsc.load_gather(s1_v, [sidx]) + plsc.load_gather(s2_v, [didx])
            selu = jnp.where(
                x > 0, SELU_LAM * x,
                (SELU_LAM * SELU_ALPHA) * (jnp.exp(x) - 1.0))
            ex = jnp.exp(selu)
            exv[0, sl] = ex
            plsc.addupdate_scatter(denv, [didx], ex)

        @pl.loop(0, EPW // 16 // 5)
        def _edge(h):
            for u in range(5):
                edge_group(h * 5 + u)

        pltpu.sync_copy(exv.at[0], ex_hbm.at[wid])
        pltpu.sync_copy(denv, den_hbm.at[cid, sid])

    return run(s_pair, src_w, dst_w)


def _sc_edge_pass(z_bf, z_f32, src_w, dst_w, ex_w):
    mesh = plsc.VectorSubcoreMesh(core_axis_name="c", subcore_axis_name="s")
    HALF = 62

    @functools.partial(
        pl.kernel,
        out_type=jax.ShapeDtypeStruct((NUM_CORES, N_NODES, Z_DIM),
                                      jnp.float32),
        mesh=mesh,
        compiler_params=_SC_PARAMS,
        scratch_types=[
            pltpu.VMEM((64, CHUNK), jnp.int32),
            pltpu.VMEM((64, CHUNK), jnp.int32),
            pltpu.VMEM((64, CHUNK), jnp.float32),
            pltpu.VMEM((CHUNK, Z_DIM), jnp.bfloat16),
            pltpu.VMEM((CHUNK, Z_DIM), jnp.bfloat16),
            pltpu.VMEM((CHUNK, Z_DIM), jnp.float32),
            pltpu.VMEM((CHUNK, Z_DIM), jnp.float32),
            pltpu.VMEM_SHARED((N_NODES, Z_DIM), jnp.float32),
            pltpu.SemaphoreType.DMA,
            pltpu.SemaphoreType.DMA,
            pltpu.SemaphoreType.DMA,
            pltpu.SemaphoreType.DMA,
        ],
    )
    def run(zb_hbm, zf_hbm, src_hbm, dst_hbm, ex_hbm, out_hbm,
            srcv, dstv, exv, gb0, gb1, fb0, fb1, table,
            gsem0, gsem1, ssem0, ssem1):
        cid = lax.axis_index("c")
        sid = lax.axis_index("s")
        wid = sid * NUM_CORES + cid

        gb = (gb0, gb1)
        fb = (fb0, fb1)
        gsem = (gsem0, gsem1)
        ssem = (ssem0, ssem1)

        pltpu.sync_copy(src_hbm.at[wid, pl.ds(0, 64)], srcv)
        pltpu.sync_copy(dst_hbm.at[wid, pl.ds(0, 64)], dstv)
        pltpu.sync_copy(ex_hbm.at[wid, pl.ds(0, 64)], exv)

        zeros16 = jnp.zeros((16,), jnp.float32)

        @pl.loop(0, CHUNK)
        def _zrows(i):
            for j in range(Z_DIM // 16):
                fb0[i, pl.ds(j * 16, 16)] = zeros16

        @pl.when(sid < NUM_SUBCORES - 1)
        def _zstripe():
            for k in range(STRIPE // CHUNK):
                pltpu.sync_copy(
                    fb0, table.at[pl.ds(sid * STRIPE + k * CHUNK, CHUNK)])
            rem = STRIPE % CHUNK
            pltpu.sync_copy(
                fb0.at[pl.ds(0, rem)],
                table.at[pl.ds(sid * STRIPE + (STRIPE // CHUNK) * CHUNK, rem)])

        @pl.when(sid == NUM_SUBCORES - 1)
        def _zstripe_last():
            base = (NUM_SUBCORES - 1) * STRIPE
            for k in range(STRIPE_LAST // CHUNK):
                pltpu.sync_copy(
                    fb0, table.at[pl.ds(base + k * CHUNK, CHUNK)])
            rem = STRIPE_LAST % CHUNK
            pltpu.sync_copy(
                fb0.at[pl.ds(0, rem)],
                table.at[pl.ds(base + (STRIPE_LAST // CHUNK) * CHUNK, rem)])

        plsc.subcore_barrier()

        def gather_start(row, b):
            pltpu.async_copy(zb_hbm.at[srcv.at[row]], gb[b], gsem[b])

        def gather_wait(b):
            pltpu.make_async_copy(
                zb_hbm.at[srcv.at[0]], gb[b], gsem[b]).wait()

        def scatter_start(row, b):
            pltpu.async_copy(
                fb[b], table.at[dstv.at[row]], ssem[b], add=True)

        def scatter_wait(b):
            pltpu.make_async_copy(
                zf_hbm.at[pl.ds(0, CHUNK)], fb[b], ssem[b]).wait()

        def compute(row, b):
            gbuf = gb[b]
            fbuf = fb[b]
            for g in range(CHUNK // 16):
                ex16 = exv[row, pl.ds(g * 16, 16)]
                for i in range(16):
                    a = ex16[i]
                    r = g * 16 + i
                    for j in range(Z_DIM // 32):
                        v = gbuf[r, pl.ds(j * 32, 32)]
                        lo, hi = plsc.unpack(v, format=plsc.PackFormat.INTERLEAVED)
                        fbuf[r, pl.ds(j * 32, 16)] = lo * a
                        fbuf[r, pl.ds(j * 32 + 16, 16)] = hi * a

        gather_start(0, 0)

        @pl.loop(0, HALF // 2)
        def _pair(i):
            r0 = i * 2
            gather_wait(0)

            @pl.when(i > 0)
            def _():
                scatter_wait(1)

            gather_start(r0 + 1, 1)
            compute(r0, 0)
            scatter_start(r0, 0)

            gather_wait(1)
            scatter_wait(0)
            gather_start(r0 + 2, 0)
            compute(r0 + 1, 1)
            scatter_start(r0 + 1, 1)

        gather_wait(0)
        scatter_wait(1)
        pltpu.sync_copy(src_hbm.at[wid, pl.ds(HALF, CPW - HALF)],
                        srcv.at[pl.ds(0, CPW - HALF)])
        pltpu.sync_copy(dst_hbm.at[wid, pl.ds(HALF, CPW - HALF)],
                        dstv.at[pl.ds(0, CPW - HALF)])
        pltpu.sync_copy(ex_hbm.at[wid, pl.ds(HALF, CPW - HALF)],
                        exv.at[pl.ds(0, CPW - HALF)])

        gather_start(1, 1)
        compute(0, 0)
        scatter_start(0, 0)

        @pl.loop(0, (CPW - HALF - 1) // 2)
        def _pair2(i):
            ra = 1 + i * 2
            gather_wait(1)
            scatter_wait(0)
            gather_start(ra + 1, 0)
            compute(ra, 1)
            scatter_start(ra, 1)

            gather_wait(0)
            scatter_wait(1)

            @pl.when(i < (CPW - HALF - 1) // 2 - 1)
            def _():
                gather_start(ra + 2, 1)

            compute(ra + 1, 0)
            scatter_start(ra + 1, 0)

        scatter_wait(0)

        plsc.subcore_barrier()

        @pl.when(sid < NUM_SUBCORES - 1)
        def _dump():
            pltpu.sync_copy(
                table.at[pl.ds(sid * STRIPE, STRIPE)],
                out_hbm.at[cid, pl.ds(sid * STRIPE, STRIPE)])

        @pl.when(sid == NUM_SUBCORES - 1)
        def _dump_last():
            base = (NUM_SUBCORES - 1) * STRIPE
            pltpu.sync_copy(
                table.at[pl.ds(base, STRIPE_LAST)],
                out_hbm.at[cid, pl.ds(base, STRIPE_LAST)])

    return run(z_bf, z_f32, src_w, dst_w, ex_w)


def _stage_combine(agg2, dens, z):
    def body(agg_ref, den_ref, z_ref, out_ref):
        acc = agg_ref[0] + agg_ref[1]
        denom = jnp.sum(den_ref[...], axis=(0, 1))[:, None]
        out_ref[...] = jnp.where(denom > 0, acc / denom, z_ref[...])

    return pl.pallas_call(
        body,
        out_shape=jax.ShapeDtypeStruct((N_NODES, Z_DIM), jnp.float32),
    )(agg2, dens, z)


_PERM = tuple(32 * g + (k // 2 if k % 2 == 0 else 16 + k // 2)
              for g in range(Z_DIM // 32) for k in range(32))


@jax.jit
def kernel(z, edge_index, W):
    wr = W.reshape(NUM_CORES, Z_DIM)
    src_w = edge_index[0].reshape(NUM_WORKERS, EPW)
    dst_w = edge_index[1].reshape(NUM_WORKERS, EPW)
    s_pair, z_bf = _stage_scores(z, wr, jnp.asarray(_PERM, dtype=jnp.int32))
    ex_w, dens = _sc_edge_weights(s_pair, src_w, dst_w)
    agg2 = _sc_edge_pass(
        z_bf, z,
        src_w.reshape(NUM_WORKERS, CPW, CHUNK),
        dst_w.reshape(NUM_WORKERS, CPW, CHUNK),
        ex_w.reshape(NUM_WORKERS, CPW, CHUNK),
        )
    return _stage_combine(agg2, dens, z)
